# Initial kernel scaffold; baseline (speedup 1.0000x reference)
import jax
import jax.numpy as jnp
from jax import lax
from jax.experimental import pallas as pl
from jax.experimental.pallas import tpu as pltpu
from jax.experimental.pallas import tpu_sc as plsc

N = 10000
E = 320000
B = 64

NTILES = 16
NCORES = 2
K = 128
EPT = E // (NCORES * NTILES)
NCHUNK = -(-(E // NTILES) // K)
EPAD = NTILES * NCHUNK * K
ACC_ROWS = 10240
ROWS_PT = ACC_ROWS // NTILES

_sc_mesh = plsc.VectorSubcoreMesh(core_axis_name="c", subcore_axis_name="s")


def _deg_body(dst_hbm, out_hbm, dstv, histv):
    c = lax.axis_index("c")
    s = lax.axis_index("s")
    wid = c * NTILES + s
    pltpu.sync_copy(dst_hbm.at[pl.ds(wid * EPT, EPT)], dstv)
    zeros = jnp.zeros((16,), jnp.float32)

    @pl.loop(0, N // 16)
    def _zero(i):
        histv[pl.ds(i * 16, 16)] = zeros

    ones = jnp.ones((16,), jnp.float32)

    @pl.loop(0, EPT // 16)
    def _hist(i):
        idx = dstv[pl.ds(i * 16, 16)]
        plsc.addupdate_scatter(histv, [idx], ones)

    pltpu.sync_copy(histv, out_hbm.at[wid])


_deg = pl.kernel(
    _deg_body,
    out_type=jax.ShapeDtypeStruct((NCORES * NTILES, N), jnp.float32),
    mesh=_sc_mesh,
    scratch_types=[
        pltpu.VMEM((EPT,), jnp.int32),
        pltpu.VMEM((N,), jnp.float32),
    ],
)


def _make_mp(nparts):
    ppc = nparts // NCORES

    def body(src_hbm, dst_hbm, tab_hbm, out_hbm, srcv, dstv, rowsv, acc, sem):
        c = lax.axis_index("c")
        s = lax.axis_index("s")
        zeros = jnp.zeros((16,), jnp.float32)

        def zero_rowsv():
            @pl.loop(0, K * 128 // 16)
            def _z(i):
                rowsv[i // 8, pl.ds((i % 8) * 16, 16)] = zeros

        def zero_acc_slice():
            @pl.loop(0, ROWS_PT // K)
            def _za(j):
                pltpu.sync_copy(rowsv, acc.at[pl.ds(s * ROWS_PT + j * K, K)])

        zero_rowsv()
        zero_acc_slice()
        plsc.subcore_barrier()

        for p_local in range(ppc):
            part = c * ppc + p_local
            tab = tab_hbm.at[part]
            out = out_hbm.at[part]

            @pl.loop(0, NCHUNK)
            def _chunk(j):
                base = (s * NCHUNK + j) * K
                pltpu.sync_copy(src_hbm.at[pl.ds(base, K)], srcv)
                pltpu.sync_copy(dst_hbm.at[pl.ds(base, K)], dstv)
                pltpu.async_copy(tab.at[srcv], rowsv, sem).wait()
                pltpu.sync_copy(rowsv, acc.at[dstv], add=True)

            plsc.subcore_barrier()

            @pl.when(s < NTILES - 1)
            def _dump_full():
                pltpu.sync_copy(
                    acc.at[pl.ds(s * ROWS_PT, ROWS_PT)],
                    out.at[pl.ds(s * ROWS_PT, ROWS_PT)],
                )

            @pl.when(s == NTILES - 1)
            def _dump_tail():
                nlast = N - (NTILES - 1) * ROWS_PT
                pltpu.sync_copy(
                    acc.at[pl.ds((NTILES - 1) * ROWS_PT, nlast)],
                    out.at[pl.ds((NTILES - 1) * ROWS_PT, nlast)],
                )

            if p_local + 1 < ppc:
                zero_rowsv()
                zero_acc_slice()
                plsc.subcore_barrier()

    return pl.kernel(
        body,
        out_type=jax.ShapeDtypeStruct((nparts, N, 128), jnp.float32),
        mesh=_sc_mesh,
        scratch_types=[
            pltpu.VMEM((K,), jnp.int32),
            pltpu.VMEM((K,), jnp.int32),
            pltpu.VMEM((K, 128), jnp.float32),
            pltpu.VMEM_SHARED((ACC_ROWS, 128), jnp.float32),
            pltpu.SemaphoreType.DMA,
        ],
    )


_mp2 = _make_mp(2)
_mp4 = _make_mp(4)


RB = 400
NRB = N // RB


def _dinv_body(h_ref, o_ref):
    o_ref[...] = lax.rsqrt(1.0 + jnp.sum(h_ref[...], axis=0))


_dinv_call = pl.pallas_call(
    _dinv_body,
    grid=(NRB,),
    in_specs=[pl.BlockSpec((NCORES * NTILES, RB, 1), lambda r: (0, r, 0))],
    out_specs=pl.BlockSpec((RB, 1), lambda r: (r, 0)),
    out_shape=jax.ShapeDtypeStruct((N, 1), jnp.float32),
)


def _mm1_body(x_ref, w_ref, dinv_ref, o_ref):
    h = jnp.dot(x_ref[...], w_ref[...], preferred_element_type=jnp.float32)
    o_ref[0] = dinv_ref[...] * h


_mm1_call = pl.pallas_call(
    _mm1_body,
    grid=(2, NRB),
    in_specs=[
        pl.BlockSpec((RB, 128), lambda p, r: (r, 0)),
        pl.BlockSpec((128, 128), lambda p, r: (0, p)),
        pl.BlockSpec((RB, 1), lambda p, r: (r, 0)),
    ],
    out_specs=pl.BlockSpec((1, RB, 128), lambda p, r: (p, r, 0)),
    out_shape=jax.ShapeDtypeStruct((2, N, 128), jnp.float32),
)


def _mm2_body(agg_ref, hs_ref, dinv_ref, b1_ref, w2_ref, o_ref):
    dinv = dinv_ref[...]
    h1 = (agg_ref[...] + hs_ref[...]) * dinv[None] + b1_ref[...]
    h1 = jnp.maximum(h1, 0.0)
    acc = jnp.dot(h1[0], w2_ref[0], preferred_element_type=jnp.float32)
    acc += jnp.dot(h1[1], w2_ref[1], preferred_element_type=jnp.float32)
    o_ref[0] = dinv * acc


_mm2_call = pl.pallas_call(
    _mm2_body,
    grid=(4, NRB),
    in_specs=[
        pl.BlockSpec((2, RB, 128), lambda p, r: (0, r, 0)),
        pl.BlockSpec((2, RB, 128), lambda p, r: (0, r, 0)),
        pl.BlockSpec((RB, 1), lambda p, r: (r, 0)),
        pl.BlockSpec((2, 1, 128), lambda p, r: (0, 0, 0)),
        pl.BlockSpec((2, 128, 128), lambda p, r: (0, 0, p)),
    ],
    out_specs=pl.BlockSpec((1, RB, 128), lambda p, r: (p, r, 0)),
    out_shape=jax.ShapeDtypeStruct((4, N, 128), jnp.float32),
)


def _pool_body(agg_ref, hs_ref, dinv_ref, b2_ref, batch_ref,
               gmax_ref, gsum_ref, cnt_ref):
    r = pl.program_id(0)

    @pl.when(r == 0)
    def _init():
        gmax_ref[...] = jnp.full(gmax_ref.shape, -jnp.inf, jnp.float32)
        gsum_ref[...] = jnp.zeros(gsum_ref.shape, jnp.float32)
        cnt_ref[...] = jnp.zeros(cnt_ref.shape, jnp.float32)

    dinv = dinv_ref[...]
    h = (agg_ref[...] + hs_ref[...]) * dinv[None] + b2_ref[...]
    seg = batch_ref[...]
    mask = seg == lax.broadcasted_iota(jnp.int32, (RB, B), 1)
    maskf = mask.astype(jnp.float32)
    cnt_ref[...] += jnp.sum(maskf, axis=0)[:, None]
    for p in range(4):
        gsum_ref[:, p] += lax.dot_general(
            maskf, h[p], (((0,), (0,)), ((), ())),
            preferred_element_type=jnp.float32)

    def body(b, _):
        mb = lax.dynamic_slice_in_dim(mask, b, 1, axis=1)
        for p in range(4):
            masked = jnp.where(mb, h[p], -jnp.inf)
            m = jnp.max(masked, axis=0, keepdims=True)
            cur = gmax_ref[pl.ds(b, 1), p]
            gmax_ref[pl.ds(b, 1), p] = jnp.maximum(cur, m)
        return 0

    lax.fori_loop(0, B, body, 0)


_pool_call = pl.pallas_call(
    _pool_body,
    grid=(NRB,),
    in_specs=[
        pl.BlockSpec((4, RB, 128), lambda r: (0, r, 0)),
        pl.BlockSpec((4, RB, 128), lambda r: (0, r, 0)),
        pl.BlockSpec((RB, 1), lambda r: (r, 0)),
        pl.BlockSpec((4, 1, 128), lambda r: (0, 0, 0)),
        pl.BlockSpec((RB, 1), lambda r: (r, 0)),
    ],
    out_specs=[
        pl.BlockSpec((B, 4, 128), lambda r: (0, 0, 0)),
        pl.BlockSpec((B, 4, 128), lambda r: (0, 0, 0)),
        pl.BlockSpec((B, 1), lambda r: (0, 0)),
    ],
    out_shape=[
        jax.ShapeDtypeStruct((B, 4, 128), jnp.float32),
        jax.ShapeDtypeStruct((B, 4, 128), jnp.float32),
        jax.ShapeDtypeStruct((B, 1), jnp.float32),
    ],
)


def _mlp_body(gmax_ref, gsum_ref, cnt_ref, w1a_ref, w1b_ref, b1_ref,
              w2_ref, b2_ref, w3_ref, b3_ref, o_ref):
    gmean = gsum_ref[...] / jnp.maximum(cnt_ref[...], 1.0)
    x1 = (jnp.dot(gmax_ref[...], w1a_ref[...], preferred_element_type=jnp.float32)
          + jnp.dot(gmean, w1b_ref[...], preferred_element_type=jnp.float32)
          + b1_ref[...])
    x2 = jnp.dot(x1, w2_ref[...], preferred_element_type=jnp.float32) + b2_ref[...]
    o_ref[...] = jnp.dot(x2, w3_ref[...], preferred_element_type=jnp.float32) + b3_ref[...]


_mlp_call = pl.pallas_call(
    _mlp_body,
    out_shape=jax.ShapeDtypeStruct((B, 1), jnp.float32),
)


def kernel(x, edge_attr, edge_index, batch_index,
           W1, b1, W2, b2, Wl1, bl1, Wl2, bl2, Wl3, bl3):
    del edge_attr
    src = edge_index[0]
    dst = edge_index[1]

    hists = _deg(dst)
    dinv = _dinv_call(hists.reshape(NCORES * NTILES, N, 1))

    npad = EPAD - E
    src_p = jnp.concatenate([src, jnp.zeros((npad,), src.dtype)])
    dst_p = jnp.concatenate([dst, jnp.full((npad,), N, dst.dtype)])

    hs1 = _mm1_call(x, W1, dinv)
    agg1 = _mp2(src_p, dst_p, hs1)
    hs2 = _mm2_call(agg1, hs1, dinv, b1.reshape(2, 1, 128),
                    W2.reshape(2, 128, 512))
    agg2 = _mp4(src_p, dst_p, hs2)
    gmax, gsum, cnt = _pool_call(agg2, hs2, dinv, b2.reshape(4, 1, 128),
                                 batch_index.reshape(N, 1))
    out = _mlp_call(gmax.reshape(B, 512), gsum.reshape(B, 512), cnt,
                    Wl1[:512], Wl1[512:], bl1.reshape(1, 512),
                    Wl2, bl2.reshape(1, 256), Wl3, bl3.reshape(1, 1))
    return out


# speedup vs baseline: 7.0657x; 7.0657x over previous
import jax
import jax.numpy as jnp
from jax import lax
from jax.experimental import pallas as pl
from jax.experimental.pallas import tpu as pltpu
from jax.experimental.pallas import tpu_sc as plsc

N = 10000
E = 320000
B = 64

NTILES = 16
NCORES = 2
K = 128
EPT = E // (NCORES * NTILES)
NCHUNK = -(-(E // NTILES) // K)
EPAD = NTILES * NCHUNK * K
ACC_ROWS = 10240
ROWS_PT = ACC_ROWS // NTILES

_sc_mesh = plsc.VectorSubcoreMesh(core_axis_name="c", subcore_axis_name="s")


_J_SPLIT = NCHUNK // 2 + 1


def _deg_body(dst_hbm, out_hbm, dstv, zbuf, onesb, acc):
    c = lax.axis_index("c")
    s = lax.axis_index("s")
    zeros = jnp.zeros((16,), jnp.float32)
    ones = jnp.ones((16,), jnp.float32)

    @pl.loop(0, K * 128 // 16)
    def _fill(i):
        zbuf[i // 8, pl.ds((i % 8) * 16, 16)] = zeros
        onesb[i // 8, pl.ds((i % 8) * 16, 16)] = ones

    @pl.loop(0, ROWS_PT // K)
    def _zero(j):
        pltpu.sync_copy(zbuf, acc.at[pl.ds(s * ROWS_PT + j * K, K)])

    plsc.subcore_barrier()

    jlo = _J_SPLIT * c
    jhi = _J_SPLIT + (NCHUNK - _J_SPLIT) * c

    @pl.loop(jlo, jhi)
    def _chunk(j):
        base = (s * NCHUNK + j) * K
        pltpu.sync_copy(dst_hbm.at[pl.ds(base, K)], dstv.at[0])
        pltpu.sync_copy(onesb, acc.at[dstv.at[0]], add=True)

    plsc.subcore_barrier()

    @pl.when(s < NTILES - 1)
    def _dump_full():
        pltpu.sync_copy(acc.at[pl.ds(s * ROWS_PT, ROWS_PT)],
                        out_hbm.at[c].at[pl.ds(s * ROWS_PT, ROWS_PT)])

    @pl.when(s == NTILES - 1)
    def _dump_tail():
        nlast = N - (NTILES - 1) * ROWS_PT
        pltpu.sync_copy(acc.at[pl.ds((NTILES - 1) * ROWS_PT, nlast)],
                        out_hbm.at[c].at[pl.ds((NTILES - 1) * ROWS_PT, nlast)])


_deg = pl.kernel(
    _deg_body,
    out_type=jax.ShapeDtypeStruct((NCORES, N, 128), jnp.float32),
    mesh=_sc_mesh,
    scratch_types=[
        pltpu.VMEM((1, K), jnp.int32),
        pltpu.VMEM((K, 128), jnp.float32),
        pltpu.VMEM((K, 128), jnp.float32),
        pltpu.VMEM_SHARED((ACC_ROWS, 128), jnp.float32),
    ],
)


def _make_mp(nparts):
    ppc = nparts // NCORES

    def body(src_hbm, dst_hbm, tab_hbm, out_hbm, idxv, rowsv, zbuf, acc, sem):
        c = lax.axis_index("c")
        s = lax.axis_index("s")
        zeros = jnp.zeros((16,), jnp.float32)

        @pl.loop(0, K * 128 // 16)
        def _zfill(i):
            zbuf[i // 8, pl.ds((i % 8) * 16, 16)] = zeros

        def zero_acc_slice():
            @pl.loop(0, ROWS_PT // K)
            def _za(j):
                pltpu.sync_copy(zbuf, acc.at[pl.ds(s * ROWS_PT + j * K, K)])

        zero_acc_slice()
        plsc.subcore_barrier()

        for p_local in range(ppc):
            part = c * ppc + p_local
            tab = tab_hbm.at[part]
            out = out_hbm.at[part]

            @pl.loop(0, NCHUNK)
            def _chunk(j):
                base = (s * NCHUNK + j) * K
                pltpu.sync_copy(src_hbm.at[pl.ds(base, K)], idxv.at[0])
                pltpu.sync_copy(dst_hbm.at[pl.ds(base, K)], idxv.at[1])
                pltpu.async_copy(tab.at[idxv.at[0]], rowsv, sem).wait()
                pltpu.sync_copy(rowsv, acc.at[idxv.at[1]], add=True)

            plsc.subcore_barrier()

            @pl.when(s < NTILES - 1)
            def _dump_full():
                pltpu.sync_copy(
                    acc.at[pl.ds(s * ROWS_PT, ROWS_PT)],
                    out.at[pl.ds(s * ROWS_PT, ROWS_PT)],
                )

            @pl.when(s == NTILES - 1)
            def _dump_tail():
                nlast = N - (NTILES - 1) * ROWS_PT
                pltpu.sync_copy(
                    acc.at[pl.ds((NTILES - 1) * ROWS_PT, nlast)],
                    out.at[pl.ds((NTILES - 1) * ROWS_PT, nlast)],
                )

            if p_local + 1 < ppc:
                zero_acc_slice()
                plsc.subcore_barrier()

    return pl.kernel(
        body,
        out_type=jax.ShapeDtypeStruct((nparts, N, 128), jnp.float32),
        mesh=_sc_mesh,
        scratch_types=[
            pltpu.VMEM((2, K), jnp.int32),
            pltpu.VMEM((K, 128), jnp.float32),
            pltpu.VMEM((K, 128), jnp.float32),
            pltpu.VMEM_SHARED((ACC_ROWS, 128), jnp.float32),
            pltpu.SemaphoreType.DMA,
        ],
    )


_mp2 = _make_mp(2)
_mp4 = _make_mp(4)


RB = 400
NRB = N // RB


def _dinv_body(h_ref, o_ref):
    h = h_ref[...]
    o_ref[...] = lax.rsqrt(1.0 + h[0, :, :1] + h[1, :, :1])


_dinv_call = pl.pallas_call(
    _dinv_body,
    grid=(NRB,),
    in_specs=[pl.BlockSpec((NCORES, RB, 128), lambda r: (0, r, 0))],
    out_specs=pl.BlockSpec((RB, 1), lambda r: (r, 0)),
    out_shape=jax.ShapeDtypeStruct((N, 1), jnp.float32),
)


def _mm1_body(x_ref, w_ref, dinv_ref, o_ref):
    h = jnp.dot(x_ref[...], w_ref[...], preferred_element_type=jnp.float32)
    o_ref[0] = dinv_ref[...] * h


_mm1_call = pl.pallas_call(
    _mm1_body,
    grid=(2, NRB),
    in_specs=[
        pl.BlockSpec((RB, 128), lambda p, r: (r, 0)),
        pl.BlockSpec((128, 128), lambda p, r: (0, p)),
        pl.BlockSpec((RB, 1), lambda p, r: (r, 0)),
    ],
    out_specs=pl.BlockSpec((1, RB, 128), lambda p, r: (p, r, 0)),
    out_shape=jax.ShapeDtypeStruct((2, N, 128), jnp.float32),
)


def _mm2_body(agg_ref, hs_ref, dinv_ref, b1_ref, w2_ref, o_ref):
    dinv = dinv_ref[...]
    h1 = (agg_ref[...] + hs_ref[...]) * dinv[None] + b1_ref[...]
    h1 = jnp.maximum(h1, 0.0)
    acc = jnp.dot(h1[0], w2_ref[0], preferred_element_type=jnp.float32)
    acc += jnp.dot(h1[1], w2_ref[1], preferred_element_type=jnp.float32)
    o_ref[0] = dinv * acc


_mm2_call = pl.pallas_call(
    _mm2_body,
    grid=(4, NRB),
    in_specs=[
        pl.BlockSpec((2, RB, 128), lambda p, r: (0, r, 0)),
        pl.BlockSpec((2, RB, 128), lambda p, r: (0, r, 0)),
        pl.BlockSpec((RB, 1), lambda p, r: (r, 0)),
        pl.BlockSpec((2, 1, 128), lambda p, r: (0, 0, 0)),
        pl.BlockSpec((2, 128, 128), lambda p, r: (0, 0, p)),
    ],
    out_specs=pl.BlockSpec((1, RB, 128), lambda p, r: (p, r, 0)),
    out_shape=jax.ShapeDtypeStruct((4, N, 128), jnp.float32),
)


def _pool_body(agg_ref, hs_ref, dinv_ref, b2_ref, batch_ref,
               gmax_ref, gsum_ref, cnt_ref):
    r = pl.program_id(0)

    @pl.when(r == 0)
    def _init():
        gmax_ref[...] = jnp.full(gmax_ref.shape, -jnp.inf, jnp.float32)
        gsum_ref[...] = jnp.zeros(gsum_ref.shape, jnp.float32)
        cnt_ref[...] = jnp.zeros(cnt_ref.shape, jnp.float32)

    dinv = dinv_ref[...]
    h = (agg_ref[...] + hs_ref[...]) * dinv[None] + b2_ref[...]
    seg = batch_ref[...]
    mask = seg == lax.broadcasted_iota(jnp.int32, (RB, B), 1)
    maskf = mask.astype(jnp.float32)
    cnt_ref[...] += jnp.sum(maskf, axis=0)[:, None]
    for p in range(4):
        gsum_ref[:, p] += lax.dot_general(
            maskf, h[p], (((0,), (0,)), ((), ())),
            precision=lax.Precision.HIGHEST,
            preferred_element_type=jnp.float32)

    def body(b, _):
        mb = seg == b
        for p in range(4):
            masked = jnp.where(mb, h[p], -jnp.inf)
            m = jnp.max(masked, axis=0, keepdims=True)
            cur = gmax_ref[pl.ds(b, 1), p]
            gmax_ref[pl.ds(b, 1), p] = jnp.maximum(cur, m)
        return 0

    lax.fori_loop(0, B, body, 0)


_pool_call = pl.pallas_call(
    _pool_body,
    grid=(NRB,),
    in_specs=[
        pl.BlockSpec((4, RB, 128), lambda r: (0, r, 0)),
        pl.BlockSpec((4, RB, 128), lambda r: (0, r, 0)),
        pl.BlockSpec((RB, 1), lambda r: (r, 0)),
        pl.BlockSpec((4, 1, 128), lambda r: (0, 0, 0)),
        pl.BlockSpec((RB, 1), lambda r: (r, 0)),
    ],
    out_specs=[
        pl.BlockSpec((B, 4, 128), lambda r: (0, 0, 0)),
        pl.BlockSpec((B, 4, 128), lambda r: (0, 0, 0)),
        pl.BlockSpec((B, 1), lambda r: (0, 0)),
    ],
    out_shape=[
        jax.ShapeDtypeStruct((B, 4, 128), jnp.float32),
        jax.ShapeDtypeStruct((B, 4, 128), jnp.float32),
        jax.ShapeDtypeStruct((B, 1), jnp.float32),
    ],
)


def _mlp_body(gmax_ref, gsum_ref, cnt_ref, w1a_ref, w1b_ref, b1_ref,
              w2_ref, b2_ref, w3_ref, b3_ref, o_ref):
    gmean = gsum_ref[...] / jnp.maximum(cnt_ref[...], 1.0)
    x1 = (jnp.dot(gmax_ref[...], w1a_ref[...], preferred_element_type=jnp.float32)
          + jnp.dot(gmean, w1b_ref[...], preferred_element_type=jnp.float32)
          + b1_ref[...])
    x2 = jnp.dot(x1, w2_ref[...], preferred_element_type=jnp.float32) + b2_ref[...]
    o_ref[...] = jnp.dot(x2, w3_ref[...], preferred_element_type=jnp.float32) + b3_ref[...]


_mlp_call = pl.pallas_call(
    _mlp_body,
    out_shape=jax.ShapeDtypeStruct((B, 1), jnp.float32),
)


def kernel(x, edge_attr, edge_index, batch_index,
           W1, b1, W2, b2, Wl1, bl1, Wl2, bl2, Wl3, bl3):
    del edge_attr
    src = edge_index[0]
    dst = edge_index[1]

    npad = EPAD - E
    src_p = jnp.concatenate([src, jnp.zeros((npad,), src.dtype)])
    pad_dst = N + (jnp.arange(npad, dtype=dst.dtype) % (ACC_ROWS - N))
    dst_p = jnp.concatenate([dst, pad_dst])

    hists = _deg(dst_p)
    dinv = _dinv_call(hists)

    hs1 = _mm1_call(x, W1, dinv)
    agg1 = _mp2(src_p, dst_p, hs1)
    hs2 = _mm2_call(agg1, hs1, dinv, b1.reshape(2, 1, 128),
                    W2.reshape(2, 128, 512))
    agg2 = _mp4(src_p, dst_p, hs2)
    gmax, gsum, cnt = _pool_call(agg2, hs2, dinv, b2.reshape(4, 1, 128),
                                 batch_index.reshape(N, 1))
    out = _mlp_call(gmax.reshape(B, 512), gsum.reshape(B, 512), cnt,
                    Wl1[:512], Wl1[512:], bl1.reshape(1, 512),
                    Wl2, bl2.reshape(1, 256), Wl3, bl3.reshape(1, 1))
    return out
